# Initial kernel scaffold; baseline (speedup 1.0000x reference)
import jax
import jax.numpy as jnp
from jax.experimental import pallas as pl


def kernel(x, edge_index, batch, W0, b0, g0, bt0, W1, b1, g1, bt1, W2, b2, g2, bt2, cW1, cb1, cW2, cb2):
    raise NotImplementedError("write your pallas kernel here")


# speedup vs baseline: 6.3417x; 6.3417x over previous
import functools

import jax
import jax.numpy as jnp
from jax import lax
from jax.experimental import pallas as pl
from jax.experimental.pallas import tpu as pltpu
from jax.experimental.pallas import tpu_sc as plsc

N = 10000
E = 160000
HD = 256
HH = HD // 2
G = 64
IN_F = 7

NC = 2
NS = 16
LANES = 16

CHUNK = 128
CPT = 80
E_PAD = NS * CHUNK * CPT
N_PAD = 10240
ROWS_PER_TILE = N_PAD // NS
HALF = N_PAD // NC
DEG_COLS = HALF // NS

BR = 1000
NB = N // BR

_MESH = plsc.VectorSubcoreMesh(
    core_axis_name="c", subcore_axis_name="s", num_cores=NC, num_subcores=NS)


def _zero_vmem_1d(ref, n):
  z = jnp.zeros((LANES,), ref.dtype)

  def body(i, _):
    ref[pl.ds(i * LANES, LANES)] = z
    return 0

  lax.fori_loop(0, n // LANES, body, 0)


def _deg_body(dst2d, deg_out, dst_v, deg16_v, acc_v, tmp_v, t128_v, deg_sh):
  c = lax.axis_index("c")
  s = lax.axis_index("s")
  base = c * HALF

  pltpu.sync_copy(dst2d.at[pl.ds(s * CPT, CPT)], dst_v)

  _zero_vmem_1d(deg16_v, LANES * HALF)

  lane_off = lax.iota(jnp.int32, LANES) * HALF
  ones16 = jnp.ones((LANES,), jnp.float32)

  def edge_chunk(j, _):
    for k in range(CHUNK // LANES):
      d16 = dst_v[j, pl.ds(k * LANES, LANES)]
      col = d16 - base
      msk = (col >= 0) & (col < HALF)
      col = jnp.where(msk, col, 0)
      plsc.addupdate_scatter(deg16_v, [lane_off + col], ones16, mask=msk)
    return 0

  lax.fori_loop(0, CPT, edge_chunk, 0)

  def red(i, _):
    sl = pl.ds(i * LANES, LANES)
    v = deg16_v[pl.ds(i * LANES, LANES)]
    for t in range(1, NS):
      v = v + deg16_v[pl.ds(t * HALF + i * LANES, LANES)]
    acc_v[sl] = v
    return 0

  lax.fori_loop(0, HALF // LANES, red, 0)

  pltpu.sync_copy(acc_v, deg_sh.at[pl.ds(s * HALF, HALF)])
  plsc.subcore_barrier()

  def reduce_block(cb, _):
    _zero_vmem_1d(tmp_v, 128)

    def addt(t, __):
      off = pl.multiple_of(t * HALF + cb * 128, 128)
      pltpu.sync_copy(deg_sh.at[pl.ds(off, 128)], t128_v)
      for i in range(128 // LANES):
        sl = pl.ds(i * LANES, LANES)
        tmp_v[sl] = tmp_v[sl] + t128_v[sl]
      return 0

    lax.fori_loop(0, NS, addt, 0)
    off_o = pl.multiple_of(c * HALF + cb * 128, 128)
    pltpu.sync_copy(tmp_v, deg_out.at[pl.ds(off_o, 128)])
    return 0

  @pl.when(s < 8)
  def _():
    lax.fori_loop(s * 3, s * 3 + 3, reduce_block, 0)

  @pl.when(s >= 8)
  def _():
    lax.fori_loop(24 + (s - 8) * 2, 24 + (s - 8) * 2 + 2, reduce_block, 0)


@functools.partial(
    pl.kernel,
    out_type=jax.ShapeDtypeStruct((N_PAD,), jnp.float32),
    mesh=_MESH,
    scratch_types=[
        pltpu.VMEM((CPT, CHUNK), jnp.int32),
        pltpu.VMEM((NS * HALF,), jnp.float32),
        pltpu.VMEM((HALF,), jnp.float32),
        pltpu.VMEM((128,), jnp.float32),
        pltpu.VMEM((128,), jnp.float32),
        pltpu.VMEM_SHARED((NS * HALF,), jnp.float32),
    ],
    compiler_params=pltpu.CompilerParams(needs_layout_passes=False),
)
def _deg_kernel(dst2d, deg_out, dst_v, deg16_v, acc_v, tmp_v, t128_v, deg_sh):
  _deg_body(dst2d, deg_out, dst_v, deg16_v, acc_v, tmp_v, t128_v, deg_sh)


def _agg_body(u_lo, u_hi, src2d, dst2d, out_lo, out_hi,
              sidx_v, didx_v, rows_v, acc_sh):
  c = lax.axis_index("c")
  s = lax.axis_index("s")

  zz = jnp.zeros((LANES,), jnp.float32)

  def zb(i, _):
    rows_v[i // (HH // LANES), pl.ds((i % (HH // LANES)) * LANES, LANES)] = zz
    return 0

  lax.fori_loop(0, CHUNK * (HH // LANES), zb, 0)
  for k in range(ROWS_PER_TILE // CHUNK):
    pltpu.sync_copy(rows_v, acc_sh.at[pl.ds(s * ROWS_PER_TILE + k * CHUNK, CHUNK)])
  plsc.subcore_barrier()

  pltpu.sync_copy(src2d.at[pl.ds(s * CPT, CPT)], sidx_v)
  pltpu.sync_copy(dst2d.at[pl.ds(s * CPT, CPT)], didx_v)

  def edge_loop(u_hbm):
    def body(j, _):
      pltpu.sync_copy(u_hbm.at[sidx_v.at[j]], rows_v)
      pltpu.sync_copy(rows_v, acc_sh.at[didx_v.at[j]], add=True)
      return 0

    lax.fori_loop(0, CPT, body, 0)

  @pl.when(c == 0)
  def _():
    edge_loop(u_lo)

  @pl.when(c == 1)
  def _():
    edge_loop(u_hi)

  plsc.subcore_barrier()

  def writeout(out_hbm):
    for k in range(ROWS_PER_TILE // CHUNK):
      sl = pl.ds(s * ROWS_PER_TILE + k * CHUNK, CHUNK)
      pltpu.sync_copy(acc_sh.at[sl], out_hbm.at[sl])

  @pl.when(c == 0)
  def _():
    writeout(out_lo)

  @pl.when(c == 1)
  def _():
    writeout(out_hi)


@functools.partial(
    pl.kernel,
    out_type=(
        jax.ShapeDtypeStruct((N_PAD, HH), jnp.float32),
        jax.ShapeDtypeStruct((N_PAD, HH), jnp.float32),
    ),
    mesh=_MESH,
    scratch_types=[
        pltpu.VMEM((CPT, CHUNK), jnp.int32),
        pltpu.VMEM((CPT, CHUNK), jnp.int32),
        pltpu.VMEM((CHUNK, HH), jnp.float32),
        pltpu.VMEM_SHARED((N_PAD, HH), jnp.float32),
    ],
    compiler_params=pltpu.CompilerParams(needs_layout_passes=False),
)
def _agg_kernel(u_lo, u_hi, src2d, dst2d, out_lo, out_hi,
                sidx_v, didx_v, rows_v, acc_sh):
  _agg_body(u_lo, u_hi, src2d, dst2d, out_lo, out_hi,
            sidx_v, didx_v, rows_v, acc_sh)


def _prep_body(deg_r, x_r, w0_r, dinv_r, ulo_r, uhi_r):
  dinv = lax.rsqrt(deg_r[...] + 1.0)
  x = x_r[...]
  w = w0_r[...]
  h = x[:, 0:1] * w[0:1, :]
  for k in range(1, IN_F):
    h = h + x[:, k:k + 1] * w[k:k + 1, :]
  u = dinv * h
  dinv_r[...] = dinv
  ulo_r[...] = u[:, :HH]
  uhi_r[...] = u[:, HH:]


def _prep_call(deg2, x, w0):
  return pl.pallas_call(
      _prep_body,
      grid=(NB,),
      in_specs=[
          pl.BlockSpec((BR, 1), lambda j: (j, 0)),
          pl.BlockSpec((BR, IN_F), lambda j: (j, 0)),
          pl.BlockSpec((IN_F, HD), lambda j: (0, 0)),
      ],
      out_specs=[
          pl.BlockSpec((BR, 1), lambda j: (j, 0)),
          pl.BlockSpec((BR, HH), lambda j: (j, 0)),
          pl.BlockSpec((BR, HH), lambda j: (j, 0)),
      ],
      out_shape=[
          jax.ShapeDtypeStruct((N, 1), jnp.float32),
          jax.ShapeDtypeStruct((N, HH), jnp.float32),
          jax.ShapeDtypeStruct((N, HH), jnp.float32),
      ],
  )(deg2, x, w0)


def _bn_relu(z, st_s, g_r, bt_r):
  m = st_s[0:1, :] * (1.0 / N)
  var = st_s[1:2, :] * (1.0 / N) - m * m
  inv = lax.rsqrt(var + 1e-5)
  return jnp.maximum(g_r[...] * (z - m) * inv + bt_r[...], 0.0)


def _stats_phase(acclo_r, acchi_r, ulo_r, uhi_r, dinv_r, b_r, z_s, st_s, j):
  dinv = dinv_r[...]
  zlo = dinv * (acclo_r[...] + ulo_r[...])
  zhi = dinv * (acchi_r[...] + uhi_r[...])
  z = jnp.concatenate([zlo, zhi], axis=1) + b_r[...]
  z_s[pl.ds(j * BR, BR), :] = z
  st = jnp.concatenate(
      [jnp.sum(z, axis=0, keepdims=True), jnp.sum(z * z, axis=0, keepdims=True)],
      axis=0)

  @pl.when(j == 0)
  def _():
    st_s[...] = st

  @pl.when(j > 0)
  def _():
    st_s[...] = st_s[...] + st


def _layer_body(acclo_r, acchi_r, ulo_r, uhi_r, dinv_r, b_r, g_r, bt_r, w_r,
                unlo_r, unhi_r, z_s, st_s):
  p = pl.program_id(0)
  j = pl.program_id(1)

  @pl.when(p == 0)
  def _():
    _stats_phase(acclo_r, acchi_r, ulo_r, uhi_r, dinv_r, b_r, z_s, st_s, j)
    unlo_r[...] = jnp.zeros((BR, HH), jnp.float32)
    unhi_r[...] = jnp.zeros((BR, HH), jnp.float32)

  @pl.when(p == 1)
  def _():
    z = z_s[pl.ds(j * BR, BR), :]
    y = _bn_relu(z, st_s, g_r, bt_r)
    h = jnp.dot(y, w_r[...], preferred_element_type=jnp.float32)
    u = dinv_r[...] * h
    unlo_r[...] = u[:, :HH]
    unhi_r[...] = u[:, HH:]


def _layer_call(acclo, acchi, ulo, uhi, dinv2, b, g, bt, w_next):
  return pl.pallas_call(
      _layer_body,
      grid=(2, NB),
      in_specs=[
          pl.BlockSpec((BR, HH), lambda p, j: (j, 0)),
          pl.BlockSpec((BR, HH), lambda p, j: (j, 0)),
          pl.BlockSpec((BR, HH), lambda p, j: (j, 0)),
          pl.BlockSpec((BR, HH), lambda p, j: (j, 0)),
          pl.BlockSpec((BR, 1), lambda p, j: (j, 0)),
          pl.BlockSpec((1, HD), lambda p, j: (0, 0)),
          pl.BlockSpec((1, HD), lambda p, j: (0, 0)),
          pl.BlockSpec((1, HD), lambda p, j: (0, 0)),
          pl.BlockSpec((HD, HD), lambda p, j: (0, 0)),
      ],
      out_specs=[
          pl.BlockSpec((BR, HH), lambda p, j: (j, 0)),
          pl.BlockSpec((BR, HH), lambda p, j: (j, 0)),
      ],
      out_shape=[
          jax.ShapeDtypeStruct((N, HH), jnp.float32),
          jax.ShapeDtypeStruct((N, HH), jnp.float32),
      ],
      scratch_shapes=[
          pltpu.VMEM((N, HD), jnp.float32),
          pltpu.VMEM((2, HD), jnp.float32),
      ],
  )(acclo, acchi, ulo, uhi, dinv2, b, g, bt, w_next)


def _final_body(acclo_r, acchi_r, ulo_r, uhi_r, dinv_r, b_r, g_r, bt_r,
                batch_r, cw1_r, cb1_r, cw2_r, cb2_r,
                out_r, z_s, st_s, emb_s, cnt_s):
  p = pl.program_id(0)
  j = pl.program_id(1)

  @pl.when(p == 0)
  def _():
    _stats_phase(acclo_r, acchi_r, ulo_r, uhi_r, dinv_r, b_r, z_s, st_s, j)

    @pl.when(j == 0)
    def _():
      out_r[...] = jnp.zeros((G, 1), jnp.float32)

  @pl.when(p == 1)
  def _():
    z = z_s[pl.ds(j * BR, BR), :]
    y = _bn_relu(z, st_s, g_r, bt_r)
    bt16 = batch_r[...].reshape(1, BR)
    gid = lax.broadcasted_iota(jnp.int32, (G, BR), 0)
    oh = (gid == bt16).astype(jnp.float32)
    part = jnp.dot(oh, y, preferred_element_type=jnp.float32)
    cpart = jnp.sum(oh, axis=1, keepdims=True)

    @pl.when(j == 0)
    def _():
      emb_s[...] = part
      cnt_s[...] = cpart

    @pl.when(j > 0)
    def _():
      emb_s[...] = emb_s[...] + part
      cnt_s[...] = cnt_s[...] + cpart

    @pl.when(j == NB - 1)
    def _():
      emb = emb_s[...] / jnp.maximum(cnt_s[...], 1.0)
      zh = jnp.maximum(
          jnp.dot(emb, cw1_r[...], preferred_element_type=jnp.float32)
          + cb1_r[...], 0.0)
      lg = jnp.dot(zh, cw2_r[...], preferred_element_type=jnp.float32) + cb2_r[...]
      out_r[...] = lg


def _final_call(acclo, acchi, ulo, uhi, dinv2, b, g, bt, batch2, cw1, cb1_2, cw2, cb2_2):
  return pl.pallas_call(
      _final_body,
      grid=(2, NB),
      in_specs=[
          pl.BlockSpec((BR, HH), lambda p, j: (j, 0)),
          pl.BlockSpec((BR, HH), lambda p, j: (j, 0)),
          pl.BlockSpec((BR, HH), lambda p, j: (j, 0)),
          pl.BlockSpec((BR, HH), lambda p, j: (j, 0)),
          pl.BlockSpec((BR, 1), lambda p, j: (j, 0)),
          pl.BlockSpec((1, HD), lambda p, j: (0, 0)),
          pl.BlockSpec((1, HD), lambda p, j: (0, 0)),
          pl.BlockSpec((1, HD), lambda p, j: (0, 0)),
          pl.BlockSpec((1, 1, BR), lambda p, j: (j, 0, 0)),
          pl.BlockSpec((HD, HH), lambda p, j: (0, 0)),
          pl.BlockSpec((1, HH), lambda p, j: (0, 0)),
          pl.BlockSpec((HH, 1), lambda p, j: (0, 0)),
          pl.BlockSpec((1, 1), lambda p, j: (0, 0)),
      ],
      out_specs=pl.BlockSpec((G, 1), lambda p, j: (0, 0)),
      out_shape=jax.ShapeDtypeStruct((G, 1), jnp.float32),
      scratch_shapes=[
          pltpu.VMEM((N, HD), jnp.float32),
          pltpu.VMEM((2, HD), jnp.float32),
          pltpu.VMEM((G, HD), jnp.float32),
          pltpu.VMEM((G, 1), jnp.float32),
      ],
  )(acclo, acchi, ulo, uhi, dinv2, b, g, bt, batch2, cw1, cb1_2, cw2, cb2_2)


def kernel(x, edge_index, batch, W0, b0, g0, bt0, W1, b1, g1, bt1,
           W2, b2, g2, bt2, cW1, cb1, cW2, cb2):
  src = edge_index[0]
  dst = edge_index[1]
  pad = E_PAD - E
  src2d = jnp.pad(src, (0, pad)).reshape(NS * CPT, CHUNK)
  dst2d = jnp.pad(dst, (0, pad), constant_values=N).reshape(NS * CPT, CHUNK)

  deg = _deg_kernel(dst2d)
  deg2 = deg[:N].reshape(N, 1)

  dinv2, u0lo, u0hi = _prep_call(deg2, x, W0)

  b0r, g0r, bt0r = b0.reshape(1, HD), g0.reshape(1, HD), bt0.reshape(1, HD)
  b1r, g1r, bt1r = b1.reshape(1, HD), g1.reshape(1, HD), bt1.reshape(1, HD)
  b2r, g2r, bt2r = b2.reshape(1, HD), g2.reshape(1, HD), bt2.reshape(1, HD)

  acc0lo, acc0hi = _agg_kernel(u0lo, u0hi, src2d, dst2d)
  u1lo, u1hi = _layer_call(acc0lo[:N], acc0hi[:N], u0lo, u0hi, dinv2,
                           b0r, g0r, bt0r, W1)

  acc1lo, acc1hi = _agg_kernel(u1lo, u1hi, src2d, dst2d)
  u2lo, u2hi = _layer_call(acc1lo[:N], acc1hi[:N], u1lo, u1hi, dinv2,
                           b1r, g1r, bt1r, W2)

  acc2lo, acc2hi = _agg_kernel(u2lo, u2hi, src2d, dst2d)
  batch2 = batch.reshape(NB, 1, BR)
  out2 = _final_call(acc2lo[:N], acc2hi[:N], u2lo, u2hi, dinv2,
                     b2r, g2r, bt2r, batch2, cW1, cb1.reshape(1, HH),
                     cW2, cb2.reshape(1, 1))
  return out2[:, 0]


# speedup vs baseline: 7.0189x; 1.1068x over previous
import functools

import jax
import jax.numpy as jnp
from jax import lax
from jax.experimental import pallas as pl
from jax.experimental.pallas import tpu as pltpu
from jax.experimental.pallas import tpu_sc as plsc

N = 10000
E = 160000
HD = 256
HH = HD // 2
G = 64
IN_F = 7

NC = 2
NS = 16
LANES = 16

CHUNK = 128
CPT = 80
E_PAD = NS * CHUNK * CPT
N_PAD = 10240
ROWS_PER_TILE = N_PAD // NS
HALF = N_PAD // NC
DEG_COLS = HALF // NS

BR = 1000
NB = N // BR

_MESH = plsc.VectorSubcoreMesh(
    core_axis_name="c", subcore_axis_name="s", num_cores=NC, num_subcores=NS)


def _zero_vmem_1d(ref, n):
  z = jnp.zeros((LANES,), ref.dtype)

  def body(i, _):
    ref[pl.ds(i * LANES, LANES)] = z
    return 0

  lax.fori_loop(0, n // LANES, body, 0)


def _deg_body(dst2d, deg_out, dst_v, deg16_v, acc_v, tmp_v, t128_v, deg_sh):
  c = lax.axis_index("c")
  s = lax.axis_index("s")
  base = c * HALF

  pltpu.sync_copy(dst2d.at[pl.ds(s * CPT, CPT)], dst_v)

  _zero_vmem_1d(deg16_v, LANES * HALF)

  lane_off = lax.iota(jnp.int32, LANES) * HALF
  ones16 = jnp.ones((LANES,), jnp.float32)

  def edge_chunk(j, _):
    for k in range(CHUNK // LANES):
      d16 = dst_v[j, pl.ds(k * LANES, LANES)]
      col = d16 - base
      msk = (col >= 0) & (col < HALF)
      col = jnp.where(msk, col, 0)
      plsc.addupdate_scatter(deg16_v, [lane_off + col], ones16, mask=msk)
    return 0

  lax.fori_loop(0, CPT, edge_chunk, 0)

  def red(i, _):
    sl = pl.ds(i * LANES, LANES)
    v = deg16_v[pl.ds(i * LANES, LANES)]
    for t in range(1, NS):
      v = v + deg16_v[pl.ds(t * HALF + i * LANES, LANES)]
    acc_v[sl] = v
    return 0

  lax.fori_loop(0, HALF // LANES, red, 0)

  pltpu.sync_copy(acc_v, deg_sh.at[pl.ds(s * HALF, HALF)])
  plsc.subcore_barrier()

  def reduce_block(cb, _):
    _zero_vmem_1d(tmp_v, 128)

    def addt(t, __):
      off = pl.multiple_of(t * HALF + cb * 128, 128)
      pltpu.sync_copy(deg_sh.at[pl.ds(off, 128)], t128_v)
      for i in range(128 // LANES):
        sl = pl.ds(i * LANES, LANES)
        tmp_v[sl] = tmp_v[sl] + t128_v[sl]
      return 0

    lax.fori_loop(0, NS, addt, 0)
    off_o = pl.multiple_of(c * HALF + cb * 128, 128)
    pltpu.sync_copy(tmp_v, deg_out.at[pl.ds(off_o, 128)])
    return 0

  @pl.when(s < 8)
  def _():
    lax.fori_loop(s * 3, s * 3 + 3, reduce_block, 0)

  @pl.when(s >= 8)
  def _():
    lax.fori_loop(24 + (s - 8) * 2, 24 + (s - 8) * 2 + 2, reduce_block, 0)


@functools.partial(
    pl.kernel,
    out_type=jax.ShapeDtypeStruct((N_PAD,), jnp.float32),
    mesh=_MESH,
    scratch_types=[
        pltpu.VMEM((CPT, CHUNK), jnp.int32),
        pltpu.VMEM((NS * HALF,), jnp.float32),
        pltpu.VMEM((HALF,), jnp.float32),
        pltpu.VMEM((128,), jnp.float32),
        pltpu.VMEM((128,), jnp.float32),
        pltpu.VMEM_SHARED((NS * HALF,), jnp.float32),
    ],
    compiler_params=pltpu.CompilerParams(needs_layout_passes=False),
)
def _deg_kernel(dst2d, deg_out, dst_v, deg16_v, acc_v, tmp_v, t128_v, deg_sh):
  _deg_body(dst2d, deg_out, dst_v, deg16_v, acc_v, tmp_v, t128_v, deg_sh)


def _agg_body(u_lo, u_hi, src2d, dst2d, out_lo, out_hi,
              sidx_v, didx_v, rows_a, rows_b, gsem_a, gsem_b,
              ssem_a, ssem_b, acc_sh):
  c = lax.axis_index("c")
  s = lax.axis_index("s")

  zz = jnp.zeros((LANES,), jnp.float32)

  def zb(i, _):
    rows_a[i // (HH // LANES), pl.ds((i % (HH // LANES)) * LANES, LANES)] = zz
    return 0

  lax.fori_loop(0, CHUNK * (HH // LANES), zb, 0)
  for k in range(ROWS_PER_TILE // CHUNK):
    pltpu.sync_copy(rows_a, acc_sh.at[pl.ds(s * ROWS_PER_TILE + k * CHUNK, CHUNK)])
  plsc.subcore_barrier()

  HCPT = CPT // 2

  def edge_loop(u_hbm):
    for half in range(2):
      base = s * CPT + half * HCPT
      pltpu.sync_copy(src2d.at[pl.ds(base, HCPT)], sidx_v)
      pltpu.sync_copy(dst2d.at[pl.ds(base, HCPT)], didx_v)
      pltpu.async_copy(u_hbm.at[sidx_v.at[0]], rows_a, gsem_a)
      pltpu.async_copy(u_hbm.at[sidx_v.at[1]], rows_b, gsem_b)

      def body(i, _):
        j = 2 * i
        pltpu.make_async_copy(u_hbm.at[sidx_v.at[j]], rows_a, gsem_a).wait()
        pltpu.async_copy(rows_a, acc_sh.at[didx_v.at[j]], ssem_a, add=True)
        pltpu.make_async_copy(u_hbm.at[sidx_v.at[j + 1]], rows_b, gsem_b).wait()
        pltpu.async_copy(rows_b, acc_sh.at[didx_v.at[j + 1]], ssem_b, add=True)

        @pl.when(i < HCPT // 2 - 1)
        def _():
          pltpu.make_async_copy(rows_a, acc_sh.at[didx_v.at[j]], ssem_a).wait()
          pltpu.async_copy(u_hbm.at[sidx_v.at[j + 2]], rows_a, gsem_a)
          pltpu.make_async_copy(rows_b, acc_sh.at[didx_v.at[j + 1]], ssem_b).wait()
          pltpu.async_copy(u_hbm.at[sidx_v.at[j + 3]], rows_b, gsem_b)

        @pl.when(i == HCPT // 2 - 1)
        def _():
          pltpu.make_async_copy(rows_a, acc_sh.at[didx_v.at[j]], ssem_a).wait()
          pltpu.make_async_copy(rows_b, acc_sh.at[didx_v.at[j + 1]], ssem_b).wait()
        return 0

      lax.fori_loop(0, HCPT // 2, body, 0)

  @pl.when(c == 0)
  def _():
    edge_loop(u_lo)

  @pl.when(c == 1)
  def _():
    edge_loop(u_hi)

  plsc.subcore_barrier()

  def writeout(out_hbm):
    for k in range(ROWS_PER_TILE // CHUNK):
      sl = pl.ds(s * ROWS_PER_TILE + k * CHUNK, CHUNK)
      pltpu.sync_copy(acc_sh.at[sl], out_hbm.at[sl])

  @pl.when(c == 0)
  def _():
    writeout(out_lo)

  @pl.when(c == 1)
  def _():
    writeout(out_hi)


@functools.partial(
    pl.kernel,
    out_type=(
        jax.ShapeDtypeStruct((N_PAD, HH), jnp.float32),
        jax.ShapeDtypeStruct((N_PAD, HH), jnp.float32),
    ),
    mesh=_MESH,
    scratch_types=[
        pltpu.VMEM((CPT // 2, CHUNK), jnp.int32),
        pltpu.VMEM((CPT // 2, CHUNK), jnp.int32),
        pltpu.VMEM((CHUNK, HH), jnp.float32),
        pltpu.VMEM((CHUNK, HH), jnp.float32),
        pltpu.SemaphoreType.DMA,
        pltpu.SemaphoreType.DMA,
        pltpu.SemaphoreType.DMA,
        pltpu.SemaphoreType.DMA,
        pltpu.VMEM_SHARED((N_PAD, HH), jnp.float32),
    ],
    compiler_params=pltpu.CompilerParams(needs_layout_passes=False),
)
def _agg_kernel(u_lo, u_hi, src2d, dst2d, out_lo, out_hi,
                sidx_v, didx_v, rows_a, rows_b, gsem_a, gsem_b,
                ssem_a, ssem_b, acc_sh):
  _agg_body(u_lo, u_hi, src2d, dst2d, out_lo, out_hi,
            sidx_v, didx_v, rows_a, rows_b, gsem_a, gsem_b,
            ssem_a, ssem_b, acc_sh)


def _prep_body(deg_r, x_r, w0_r, dinv_r, ulo_r, uhi_r):
  dinv = lax.rsqrt(deg_r[...] + 1.0)
  x = x_r[...]
  w = w0_r[...]
  h = x[:, 0:1] * w[0:1, :]
  for k in range(1, IN_F):
    h = h + x[:, k:k + 1] * w[k:k + 1, :]
  u = dinv * h
  dinv_r[...] = dinv
  ulo_r[...] = u[:, :HH]
  uhi_r[...] = u[:, HH:]


def _prep_call(deg2, x, w0):
  return pl.pallas_call(
      _prep_body,
      grid=(NB,),
      in_specs=[
          pl.BlockSpec((BR, 1), lambda j: (j, 0)),
          pl.BlockSpec((BR, IN_F), lambda j: (j, 0)),
          pl.BlockSpec((IN_F, HD), lambda j: (0, 0)),
      ],
      out_specs=[
          pl.BlockSpec((BR, 1), lambda j: (j, 0)),
          pl.BlockSpec((BR, HH), lambda j: (j, 0)),
          pl.BlockSpec((BR, HH), lambda j: (j, 0)),
      ],
      out_shape=[
          jax.ShapeDtypeStruct((N, 1), jnp.float32),
          jax.ShapeDtypeStruct((N, HH), jnp.float32),
          jax.ShapeDtypeStruct((N, HH), jnp.float32),
      ],
  )(deg2, x, w0)


def _bn_relu(z, st_s, g_r, bt_r):
  m = st_s[0:1, :] * (1.0 / N)
  var = st_s[1:2, :] * (1.0 / N) - m * m
  inv = lax.rsqrt(var + 1e-5)
  return jnp.maximum(g_r[...] * (z - m) * inv + bt_r[...], 0.0)


def _stats_phase(acclo_r, acchi_r, ulo_r, uhi_r, dinv_r, b_r, z_s, st_s, j):
  dinv = dinv_r[...]
  zlo = dinv * (acclo_r[...] + ulo_r[...])
  zhi = dinv * (acchi_r[...] + uhi_r[...])
  z = jnp.concatenate([zlo, zhi], axis=1) + b_r[...]
  z_s[pl.ds(j * BR, BR), :] = z
  st = jnp.concatenate(
      [jnp.sum(z, axis=0, keepdims=True), jnp.sum(z * z, axis=0, keepdims=True)],
      axis=0)

  @pl.when(j == 0)
  def _():
    st_s[...] = st

  @pl.when(j > 0)
  def _():
    st_s[...] = st_s[...] + st


def _layer_body(acclo_r, acchi_r, ulo_r, uhi_r, dinv_r, b_r, g_r, bt_r, w_r,
                unlo_r, unhi_r, z_s, st_s):
  p = pl.program_id(0)
  j = pl.program_id(1)

  @pl.when(p == 0)
  def _():
    _stats_phase(acclo_r, acchi_r, ulo_r, uhi_r, dinv_r, b_r, z_s, st_s, j)

  @pl.when(p == 1)
  def _():
    z = z_s[pl.ds(j * BR, BR), :]
    y = _bn_relu(z, st_s, g_r, bt_r)
    h = jnp.dot(y, w_r[...], preferred_element_type=jnp.float32)
    u = dinv_r[...] * h
    unlo_r[...] = u[:, :HH]
    unhi_r[...] = u[:, HH:]


def _layer_call(acclo, acchi, ulo, uhi, dinv2, b, g, bt, w_next):
  return pl.pallas_call(
      _layer_body,
      grid=(2, NB),
      in_specs=[
          pl.BlockSpec((BR, HH), lambda p, j: (j * (1 - p), 0)),
          pl.BlockSpec((BR, HH), lambda p, j: (j * (1 - p), 0)),
          pl.BlockSpec((BR, HH), lambda p, j: (j * (1 - p), 0)),
          pl.BlockSpec((BR, HH), lambda p, j: (j * (1 - p), 0)),
          pl.BlockSpec((BR, 1), lambda p, j: (j, 0)),
          pl.BlockSpec((1, HD), lambda p, j: (0, 0)),
          pl.BlockSpec((1, HD), lambda p, j: (0, 0)),
          pl.BlockSpec((1, HD), lambda p, j: (0, 0)),
          pl.BlockSpec((HD, HD), lambda p, j: (0, 0)),
      ],
      out_specs=[
          pl.BlockSpec((BR, HH), lambda p, j: (j * p, 0)),
          pl.BlockSpec((BR, HH), lambda p, j: (j * p, 0)),
      ],
      out_shape=[
          jax.ShapeDtypeStruct((N, HH), jnp.float32),
          jax.ShapeDtypeStruct((N, HH), jnp.float32),
      ],
      scratch_shapes=[
          pltpu.VMEM((N, HD), jnp.float32),
          pltpu.VMEM((2, HD), jnp.float32),
      ],
  )(acclo, acchi, ulo, uhi, dinv2, b, g, bt, w_next)


def _final_body(acclo_r, acchi_r, ulo_r, uhi_r, dinv_r, b_r, g_r, bt_r,
                batch_r, cw1_r, cb1_r, cw2_r, cb2_r,
                out_r, z_s, st_s, emb_s, cnt_s):
  p = pl.program_id(0)
  j = pl.program_id(1)

  @pl.when(p == 0)
  def _():
    _stats_phase(acclo_r, acchi_r, ulo_r, uhi_r, dinv_r, b_r, z_s, st_s, j)

    @pl.when(j == 0)
    def _():
      out_r[...] = jnp.zeros((G, 1), jnp.float32)

  @pl.when(p == 1)
  def _():
    z = z_s[pl.ds(j * BR, BR), :]
    y = _bn_relu(z, st_s, g_r, bt_r)
    bt16 = batch_r[...].reshape(1, BR)
    gid = lax.broadcasted_iota(jnp.int32, (G, BR), 0)
    oh = (gid == bt16).astype(jnp.float32)
    part = jnp.dot(oh, y, preferred_element_type=jnp.float32)
    cpart = jnp.sum(oh, axis=1, keepdims=True)

    @pl.when(j == 0)
    def _():
      emb_s[...] = part
      cnt_s[...] = cpart

    @pl.when(j > 0)
    def _():
      emb_s[...] = emb_s[...] + part
      cnt_s[...] = cnt_s[...] + cpart

    @pl.when(j == NB - 1)
    def _():
      emb = emb_s[...] / jnp.maximum(cnt_s[...], 1.0)
      zh = jnp.maximum(
          jnp.dot(emb, cw1_r[...], preferred_element_type=jnp.float32)
          + cb1_r[...], 0.0)
      lg = jnp.dot(zh, cw2_r[...], preferred_element_type=jnp.float32) + cb2_r[...]
      out_r[...] = lg


def _final_call(acclo, acchi, ulo, uhi, dinv2, b, g, bt, batch2, cw1, cb1_2, cw2, cb2_2):
  return pl.pallas_call(
      _final_body,
      grid=(2, NB),
      in_specs=[
          pl.BlockSpec((BR, HH), lambda p, j: (j * (1 - p), 0)),
          pl.BlockSpec((BR, HH), lambda p, j: (j * (1 - p), 0)),
          pl.BlockSpec((BR, HH), lambda p, j: (j * (1 - p), 0)),
          pl.BlockSpec((BR, HH), lambda p, j: (j * (1 - p), 0)),
          pl.BlockSpec((BR, 1), lambda p, j: (j, 0)),
          pl.BlockSpec((1, HD), lambda p, j: (0, 0)),
          pl.BlockSpec((1, HD), lambda p, j: (0, 0)),
          pl.BlockSpec((1, HD), lambda p, j: (0, 0)),
          pl.BlockSpec((1, 1, BR), lambda p, j: (j * p, 0, 0)),
          pl.BlockSpec((HD, HH), lambda p, j: (0, 0)),
          pl.BlockSpec((1, HH), lambda p, j: (0, 0)),
          pl.BlockSpec((HH, 1), lambda p, j: (0, 0)),
          pl.BlockSpec((1, 1), lambda p, j: (0, 0)),
      ],
      out_specs=pl.BlockSpec((G, 1), lambda p, j: (0, 0)),
      out_shape=jax.ShapeDtypeStruct((G, 1), jnp.float32),
      scratch_shapes=[
          pltpu.VMEM((N, HD), jnp.float32),
          pltpu.VMEM((2, HD), jnp.float32),
          pltpu.VMEM((G, HD), jnp.float32),
          pltpu.VMEM((G, 1), jnp.float32),
      ],
  )(acclo, acchi, ulo, uhi, dinv2, b, g, bt, batch2, cw1, cb1_2, cw2, cb2_2)


def kernel(x, edge_index, batch, W0, b0, g0, bt0, W1, b1, g1, bt1,
           W2, b2, g2, bt2, cW1, cb1, cW2, cb2):
  src = edge_index[0]
  dst = edge_index[1]
  pad = E_PAD - E
  src2d = jnp.pad(src, (0, pad)).reshape(NS * CPT, CHUNK)
  dst2d = jnp.pad(dst, (0, pad), constant_values=N).reshape(NS * CPT, CHUNK)

  deg = _deg_kernel(dst2d)
  deg2 = deg[:N].reshape(N, 1)

  dinv2, u0lo, u0hi = _prep_call(deg2, x, W0)

  b0r, g0r, bt0r = b0.reshape(1, HD), g0.reshape(1, HD), bt0.reshape(1, HD)
  b1r, g1r, bt1r = b1.reshape(1, HD), g1.reshape(1, HD), bt1.reshape(1, HD)
  b2r, g2r, bt2r = b2.reshape(1, HD), g2.reshape(1, HD), bt2.reshape(1, HD)

  acc0lo, acc0hi = _agg_kernel(u0lo, u0hi, src2d, dst2d)
  u1lo, u1hi = _layer_call(acc0lo[:N], acc0hi[:N], u0lo, u0hi, dinv2,
                           b0r, g0r, bt0r, W1)

  acc1lo, acc1hi = _agg_kernel(u1lo, u1hi, src2d, dst2d)
  u2lo, u2hi = _layer_call(acc1lo[:N], acc1hi[:N], u1lo, u1hi, dinv2,
                           b1r, g1r, bt1r, W2)

  acc2lo, acc2hi = _agg_kernel(u2lo, u2hi, src2d, dst2d)
  batch2 = batch.reshape(NB, 1, BR)
  out2 = _final_call(acc2lo[:N], acc2hi[:N], u2lo, u2hi, dinv2,
                     b2r, g2r, bt2r, batch2, cW1, cb1.reshape(1, HH),
                     cW2, cb2.reshape(1, 1))
  return out2[:, 0]


# speedup vs baseline: 7.5652x; 1.0778x over previous
import functools

import jax
import jax.numpy as jnp
from jax import lax
from jax.experimental import pallas as pl
from jax.experimental.pallas import tpu as pltpu
from jax.experimental.pallas import tpu_sc as plsc

N = 10000
E = 160000
HD = 256
HH = HD // 2
G = 64
IN_F = 7

NC = 2
NS = 16
LANES = 16

CHUNK = 128
CPT = 80
E_PAD = NS * CHUNK * CPT
N_PAD = 10240
ROWS_PER_TILE = N_PAD // NS
HALF = N_PAD // NC
DEG_COLS = HALF // NS

BR = 1000
NB = N // BR

_MESH = plsc.VectorSubcoreMesh(
    core_axis_name="c", subcore_axis_name="s", num_cores=NC, num_subcores=NS)


def _zero_vmem_1d(ref, n):
  z = jnp.zeros((LANES,), ref.dtype)

  def body(i, _):
    ref[pl.ds(i * LANES, LANES)] = z
    return 0

  lax.fori_loop(0, n // LANES, body, 0)


def _deg_body(dst2d, deg_out, dst_v, deg16_v, acc_v, tmp_v, t128_v, deg_sh):
  c = lax.axis_index("c")
  s = lax.axis_index("s")
  base = c * HALF

  pltpu.sync_copy(dst2d.at[pl.ds(s * CPT, CPT)], dst_v)

  _zero_vmem_1d(deg16_v, LANES * HALF)

  lane_off = lax.iota(jnp.int32, LANES) * HALF
  ones16 = jnp.ones((LANES,), jnp.float32)

  def edge_chunk(j, _):
    for k in range(CHUNK // LANES):
      d16 = dst_v[j, pl.ds(k * LANES, LANES)]
      col = d16 - base
      msk = (col >= 0) & (col < HALF)
      col = jnp.where(msk, col, 0)
      plsc.addupdate_scatter(deg16_v, [lane_off + col], ones16, mask=msk)
    return 0

  lax.fori_loop(0, CPT, edge_chunk, 0)

  def red(i, _):
    sl = pl.ds(i * LANES, LANES)
    v = deg16_v[pl.ds(i * LANES, LANES)]
    for t in range(1, NS):
      v = v + deg16_v[pl.ds(t * HALF + i * LANES, LANES)]
    acc_v[sl] = v
    return 0

  lax.fori_loop(0, HALF // LANES, red, 0)

  pltpu.sync_copy(acc_v, deg_sh.at[pl.ds(s * HALF, HALF)])
  plsc.subcore_barrier()

  def reduce_block(cb, _):
    _zero_vmem_1d(tmp_v, 128)

    def addt(t, __):
      off = pl.multiple_of(t * HALF + cb * 128, 128)
      pltpu.sync_copy(deg_sh.at[pl.ds(off, 128)], t128_v)
      for i in range(128 // LANES):
        sl = pl.ds(i * LANES, LANES)
        tmp_v[sl] = tmp_v[sl] + t128_v[sl]
      return 0

    lax.fori_loop(0, NS, addt, 0)
    off_o = pl.multiple_of(c * HALF + cb * 128, 128)
    pltpu.sync_copy(tmp_v, deg_out.at[pl.ds(off_o, 128)])
    return 0

  @pl.when(s < 8)
  def _():
    lax.fori_loop(s * 3, s * 3 + 3, reduce_block, 0)

  @pl.when(s >= 8)
  def _():
    lax.fori_loop(24 + (s - 8) * 2, 24 + (s - 8) * 2 + 2, reduce_block, 0)


@functools.partial(
    pl.kernel,
    out_type=jax.ShapeDtypeStruct((N_PAD,), jnp.float32),
    mesh=_MESH,
    scratch_types=[
        pltpu.VMEM((CPT, CHUNK), jnp.int32),
        pltpu.VMEM((NS * HALF,), jnp.float32),
        pltpu.VMEM((HALF,), jnp.float32),
        pltpu.VMEM((128,), jnp.float32),
        pltpu.VMEM((128,), jnp.float32),
        pltpu.VMEM_SHARED((NS * HALF,), jnp.float32),
    ],
    compiler_params=pltpu.CompilerParams(needs_layout_passes=False),
)
def _deg_kernel(dst2d, deg_out, dst_v, deg16_v, acc_v, tmp_v, t128_v, deg_sh):
  _deg_body(dst2d, deg_out, dst_v, deg16_v, acc_v, tmp_v, t128_v, deg_sh)


def _agg_body(u_lo, u_hi, src2d, dst2d, out_lo, out_hi,
              sidx_v, didx_v, rows_a, rows_b, gsem_a, gsem_b,
              ssem_a, ssem_b, acc_sh):
  c = lax.axis_index("c")
  s = lax.axis_index("s")

  zz = jnp.zeros((LANES,), jnp.float32)

  def zb(i, _):
    rows_a[i // (HH // LANES), pl.ds((i % (HH // LANES)) * LANES, LANES)] = zz
    return 0

  lax.fori_loop(0, CHUNK * (HH // LANES), zb, 0)
  for k in range(ROWS_PER_TILE // CHUNK):
    pltpu.sync_copy(rows_a, acc_sh.at[pl.ds(s * ROWS_PER_TILE + k * CHUNK, CHUNK)])
  plsc.subcore_barrier()

  HCPT = CPT // 2

  def edge_loop(u_hbm):
    for half in range(2):
      base = s * CPT + half * HCPT
      pltpu.sync_copy(src2d.at[pl.ds(base, HCPT)], sidx_v)
      pltpu.sync_copy(dst2d.at[pl.ds(base, HCPT)], didx_v)
      pltpu.async_copy(u_hbm.at[sidx_v.at[0]], rows_a, gsem_a)

      def body(i, _):
        j = 2 * i
        pltpu.async_copy(u_hbm.at[sidx_v.at[j + 1]], rows_b, gsem_b)
        pltpu.make_async_copy(u_hbm.at[sidx_v.at[j]], rows_a, gsem_a).wait()
        pltpu.sync_copy(rows_a, acc_sh.at[didx_v.at[j]], add=True)

        @pl.when(i < HCPT // 2 - 1)
        def _():
          pltpu.async_copy(u_hbm.at[sidx_v.at[j + 2]], rows_a, gsem_a)

        pltpu.make_async_copy(u_hbm.at[sidx_v.at[j + 1]], rows_b, gsem_b).wait()
        pltpu.sync_copy(rows_b, acc_sh.at[didx_v.at[j + 1]], add=True)
        return 0

      lax.fori_loop(0, HCPT // 2, body, 0)

  @pl.when(c == 0)
  def _():
    edge_loop(u_lo)

  @pl.when(c == 1)
  def _():
    edge_loop(u_hi)

  plsc.subcore_barrier()

  def writeout(out_hbm):
    for k in range(ROWS_PER_TILE // CHUNK):
      sl = pl.ds(s * ROWS_PER_TILE + k * CHUNK, CHUNK)
      pltpu.sync_copy(acc_sh.at[sl], out_hbm.at[sl])

  @pl.when(c == 0)
  def _():
    writeout(out_lo)

  @pl.when(c == 1)
  def _():
    writeout(out_hi)


@functools.partial(
    pl.kernel,
    out_type=(
        jax.ShapeDtypeStruct((N_PAD, HH), jnp.float32),
        jax.ShapeDtypeStruct((N_PAD, HH), jnp.float32),
    ),
    mesh=_MESH,
    scratch_types=[
        pltpu.VMEM((CPT // 2, CHUNK), jnp.int32),
        pltpu.VMEM((CPT // 2, CHUNK), jnp.int32),
        pltpu.VMEM((CHUNK, HH), jnp.float32),
        pltpu.VMEM((CHUNK, HH), jnp.float32),
        pltpu.SemaphoreType.DMA,
        pltpu.SemaphoreType.DMA,
        pltpu.SemaphoreType.DMA,
        pltpu.SemaphoreType.DMA,
        pltpu.VMEM_SHARED((N_PAD, HH), jnp.float32),
    ],
    compiler_params=pltpu.CompilerParams(needs_layout_passes=False),
)
def _agg_kernel(u_lo, u_hi, src2d, dst2d, out_lo, out_hi,
                sidx_v, didx_v, rows_a, rows_b, gsem_a, gsem_b,
                ssem_a, ssem_b, acc_sh):
  _agg_body(u_lo, u_hi, src2d, dst2d, out_lo, out_hi,
            sidx_v, didx_v, rows_a, rows_b, gsem_a, gsem_b,
            ssem_a, ssem_b, acc_sh)


def _prep_body(deg_r, x_r, w0_r, dinv_r, ulo_r, uhi_r):
  dinv = lax.rsqrt(deg_r[...] + 1.0)
  h = jnp.dot(x_r[...], w0_r[...], preferred_element_type=jnp.float32)
  u = dinv * h
  dinv_r[...] = dinv
  ulo_r[...] = u[:, :HH]
  uhi_r[...] = u[:, HH:]


def _prep_call(deg2, x, w0):
  return pl.pallas_call(
      _prep_body,
      grid=(NB,),
      in_specs=[
          pl.BlockSpec((BR, 1), lambda j: (j, 0)),
          pl.BlockSpec((BR, IN_F), lambda j: (j, 0)),
          pl.BlockSpec((IN_F, HD), lambda j: (0, 0)),
      ],
      out_specs=[
          pl.BlockSpec((BR, 1), lambda j: (j, 0)),
          pl.BlockSpec((BR, HH), lambda j: (j, 0)),
          pl.BlockSpec((BR, HH), lambda j: (j, 0)),
      ],
      out_shape=[
          jax.ShapeDtypeStruct((N, 1), jnp.float32),
          jax.ShapeDtypeStruct((N, HH), jnp.float32),
          jax.ShapeDtypeStruct((N, HH), jnp.float32),
      ],
  )(deg2, x, w0)


def _bn_relu(z, st_s, g_r, bt_r):
  m = st_s[0:1, :] * (1.0 / N)
  var = st_s[1:2, :] * (1.0 / N) - m * m
  inv = lax.rsqrt(var + 1e-5)
  return jnp.maximum(g_r[...] * (z - m) * inv + bt_r[...], 0.0)


def _stats_phase(acclo_r, acchi_r, ulo_r, uhi_r, dinv_r, b_r, z_s, st_s, j):
  dinv = dinv_r[...]
  zlo = dinv * (acclo_r[...] + ulo_r[...])
  zhi = dinv * (acchi_r[...] + uhi_r[...])
  z = jnp.concatenate([zlo, zhi], axis=1) + b_r[...]
  z_s[pl.ds(j * BR, BR), :] = z
  st = jnp.concatenate(
      [jnp.sum(z, axis=0, keepdims=True), jnp.sum(z * z, axis=0, keepdims=True)],
      axis=0)

  @pl.when(j == 0)
  def _():
    st_s[...] = st

  @pl.when(j > 0)
  def _():
    st_s[...] = st_s[...] + st


def _layer_body(acclo_r, acchi_r, ulo_r, uhi_r, dinv_r, b_r, g_r, bt_r, w_r,
                unlo_r, unhi_r, z_s, st_s):
  p = pl.program_id(0)
  j = pl.program_id(1)

  @pl.when(p == 0)
  def _():
    _stats_phase(acclo_r, acchi_r, ulo_r, uhi_r, dinv_r, b_r, z_s, st_s, j)

  @pl.when(p == 1)
  def _():
    z = z_s[pl.ds(j * BR, BR), :]
    y = _bn_relu(z, st_s, g_r, bt_r)
    h = jnp.dot(y, w_r[...], preferred_element_type=jnp.float32)
    u = dinv_r[...] * h
    unlo_r[...] = u[:, :HH]
    unhi_r[...] = u[:, HH:]


def _layer_call(acclo, acchi, ulo, uhi, dinv2, b, g, bt, w_next):
  return pl.pallas_call(
      _layer_body,
      grid=(2, NB),
      in_specs=[
          pl.BlockSpec((BR, HH), lambda p, j: (j * (1 - p), 0)),
          pl.BlockSpec((BR, HH), lambda p, j: (j * (1 - p), 0)),
          pl.BlockSpec((BR, HH), lambda p, j: (j * (1 - p), 0)),
          pl.BlockSpec((BR, HH), lambda p, j: (j * (1 - p), 0)),
          pl.BlockSpec((BR, 1), lambda p, j: (j, 0)),
          pl.BlockSpec((1, HD), lambda p, j: (0, 0)),
          pl.BlockSpec((1, HD), lambda p, j: (0, 0)),
          pl.BlockSpec((1, HD), lambda p, j: (0, 0)),
          pl.BlockSpec((HD, HD), lambda p, j: (0, 0)),
      ],
      out_specs=[
          pl.BlockSpec((BR, HH), lambda p, j: (j * p, 0)),
          pl.BlockSpec((BR, HH), lambda p, j: (j * p, 0)),
      ],
      out_shape=[
          jax.ShapeDtypeStruct((N, HH), jnp.float32),
          jax.ShapeDtypeStruct((N, HH), jnp.float32),
      ],
      scratch_shapes=[
          pltpu.VMEM((N, HD), jnp.float32),
          pltpu.VMEM((2, HD), jnp.float32),
      ],
  )(acclo, acchi, ulo, uhi, dinv2, b, g, bt, w_next)


def _final_body(acclo_r, acchi_r, ulo_r, uhi_r, dinv_r, b_r, g_r, bt_r,
                batch_r, cw1_r, cb1_r, cw2_r, cb2_r,
                out_r, z_s, st_s, emb_s, cnt_s):
  p = pl.program_id(0)
  j = pl.program_id(1)

  @pl.when(p == 0)
  def _():
    _stats_phase(acclo_r, acchi_r, ulo_r, uhi_r, dinv_r, b_r, z_s, st_s, j)

    @pl.when(j == 0)
    def _():
      out_r[...] = jnp.zeros((G, 1), jnp.float32)

  @pl.when(p == 1)
  def _():
    z = z_s[pl.ds(j * BR, BR), :]
    y = _bn_relu(z, st_s, g_r, bt_r)
    bt16 = batch_r[...].reshape(1, BR)
    gid = lax.broadcasted_iota(jnp.int32, (G, BR), 0)
    oh = (gid == bt16).astype(jnp.float32)
    part = jnp.dot(oh, y, precision=lax.Precision.HIGHEST,
                   preferred_element_type=jnp.float32)
    cpart = jnp.sum(oh, axis=1, keepdims=True)

    @pl.when(j == 0)
    def _():
      emb_s[...] = part
      cnt_s[...] = cpart

    @pl.when(j > 0)
    def _():
      emb_s[...] = emb_s[...] + part
      cnt_s[...] = cnt_s[...] + cpart

    @pl.when(j == NB - 1)
    def _():
      emb = emb_s[...] / jnp.maximum(cnt_s[...], 1.0)
      zh = jnp.maximum(
          jnp.dot(emb, cw1_r[...], preferred_element_type=jnp.float32)
          + cb1_r[...], 0.0)
      lg = jnp.dot(zh, cw2_r[...], preferred_element_type=jnp.float32) + cb2_r[...]
      out_r[...] = lg


def _final_call(acclo, acchi, ulo, uhi, dinv2, b, g, bt, batch2, cw1, cb1_2, cw2, cb2_2):
  return pl.pallas_call(
      _final_body,
      grid=(2, NB),
      in_specs=[
          pl.BlockSpec((BR, HH), lambda p, j: (j * (1 - p), 0)),
          pl.BlockSpec((BR, HH), lambda p, j: (j * (1 - p), 0)),
          pl.BlockSpec((BR, HH), lambda p, j: (j * (1 - p), 0)),
          pl.BlockSpec((BR, HH), lambda p, j: (j * (1 - p), 0)),
          pl.BlockSpec((BR, 1), lambda p, j: (j, 0)),
          pl.BlockSpec((1, HD), lambda p, j: (0, 0)),
          pl.BlockSpec((1, HD), lambda p, j: (0, 0)),
          pl.BlockSpec((1, HD), lambda p, j: (0, 0)),
          pl.BlockSpec((1, 1, BR), lambda p, j: (j * p, 0, 0)),
          pl.BlockSpec((HD, HH), lambda p, j: (0, 0)),
          pl.BlockSpec((1, HH), lambda p, j: (0, 0)),
          pl.BlockSpec((HH, 1), lambda p, j: (0, 0)),
          pl.BlockSpec((1, 1), lambda p, j: (0, 0)),
      ],
      out_specs=pl.BlockSpec((G, 1), lambda p, j: (0, 0)),
      out_shape=jax.ShapeDtypeStruct((G, 1), jnp.float32),
      scratch_shapes=[
          pltpu.VMEM((N, HD), jnp.float32),
          pltpu.VMEM((2, HD), jnp.float32),
          pltpu.VMEM((G, HD), jnp.float32),
          pltpu.VMEM((G, 1), jnp.float32),
      ],
  )(acclo, acchi, ulo, uhi, dinv2, b, g, bt, batch2, cw1, cb1_2, cw2, cb2_2)


def kernel(x, edge_index, batch, W0, b0, g0, bt0, W1, b1, g1, bt1,
           W2, b2, g2, bt2, cW1, cb1, cW2, cb2):
  src = edge_index[0]
  dst = edge_index[1]
  pad = E_PAD - E
  src2d = jnp.pad(src, (0, pad)).reshape(NS * CPT, CHUNK)
  dst2d = jnp.pad(dst, (0, pad), constant_values=N).reshape(NS * CPT, CHUNK)

  deg = _deg_kernel(dst2d)
  deg2 = deg[:N].reshape(N, 1)

  dinv2, u0lo, u0hi = _prep_call(deg2, x, W0)

  b0r, g0r, bt0r = b0.reshape(1, HD), g0.reshape(1, HD), bt0.reshape(1, HD)
  b1r, g1r, bt1r = b1.reshape(1, HD), g1.reshape(1, HD), bt1.reshape(1, HD)
  b2r, g2r, bt2r = b2.reshape(1, HD), g2.reshape(1, HD), bt2.reshape(1, HD)

  acc0lo, acc0hi = _agg_kernel(u0lo, u0hi, src2d, dst2d)
  u1lo, u1hi = _layer_call(acc0lo[:N], acc0hi[:N], u0lo, u0hi, dinv2,
                           b0r, g0r, bt0r, W1)

  acc1lo, acc1hi = _agg_kernel(u1lo, u1hi, src2d, dst2d)
  u2lo, u2hi = _layer_call(acc1lo[:N], acc1hi[:N], u1lo, u1hi, dinv2,
                           b1r, g1r, bt1r, W2)

  acc2lo, acc2hi = _agg_kernel(u2lo, u2hi, src2d, dst2d)
  batch2 = batch.reshape(NB, 1, BR)
  out2 = _final_call(acc2lo[:N], acc2hi[:N], u2lo, u2hi, dinv2,
                     b2r, g2r, bt2r, batch2, cW1, cb1.reshape(1, HH),
                     cW2, cb2.reshape(1, 1))
  return out2[:, 0]
